# Initial kernel scaffold; baseline (speedup 1.0000x reference)
import jax
import jax.numpy as jnp
from jax.experimental import pallas as pl


def kernel(x, y):
    raise NotImplementedError("write your pallas kernel here")


# speedup vs baseline: 53.1167x; 53.1167x over previous
import functools

import jax
import jax.numpy as jnp
from jax import lax
from jax.experimental import pallas as pl

_N_CLASSES = 100000
_K = 5
_TAU = 1.0
_BATCH = 128
_BR = 8


def _entr_loss_body(x_ref, y_ref, out_ref):
    rows = x_ref[...] * (1.0 / _TAU)
    yv = y_ref[...]
    col = lax.broadcasted_iota(jnp.int32, rows.shape, 1)
    is_y = col == yv

    fy = jnp.sum(jnp.where(is_y, rows, 0.0), axis=1, keepdims=True)

    z = jnp.minimum(rows - fy, 80.0)
    e = jnp.exp(z)
    sum_all = jnp.sum(e, axis=1, keepdims=True)

    cnt_gt = jnp.sum(jnp.where(rows > fy, 1.0, 0.0), axis=1, keepdims=True)
    cnt_eqb = jnp.sum(
        jnp.where((rows == fy) & (col < yv), 1.0, 0.0), axis=1, keepdims=True)
    rank = cnt_gt + cnt_eqb

    tops = jnp.zeros_like(fy)
    remaining = jnp.full_like(fy, float(_K))
    thresh = jnp.full_like(fy, jnp.inf)
    neg_inf = jnp.float32(-jnp.inf)
    for _ in range(_K):
        m = jnp.max(jnp.where(rows < thresh, rows, neg_inf),
                    axis=1, keepdims=True)
        c = jnp.sum(jnp.where(rows == m, 1.0, 0.0), axis=1, keepdims=True)
        take = jnp.minimum(remaining, c)
        tops += take * jnp.exp(jnp.minimum(m - fy, 80.0))
        remaining -= take
        thresh = m

    tail = sum_all - tops - jnp.where(rank >= float(_K), 1.0, 0.0)
    losses = jnp.log(1.0 + tail)

    @pl.when(pl.program_id(0) == 0)
    def _init():
        out_ref[...] = jnp.zeros_like(out_ref)

    out_ref[...] += jnp.sum(losses).reshape(1, 1) * (1.0 / _BATCH)


@jax.jit
def kernel(x, y):
    grid = _BATCH // _BR
    y2 = y.reshape(_BATCH, 1)
    out = pl.pallas_call(
        _entr_loss_body,
        grid=(grid,),
        in_specs=[
            pl.BlockSpec((_BR, _N_CLASSES), lambda i: (i, 0)),
            pl.BlockSpec((_BR, 1), lambda i: (i, 0)),
        ],
        out_specs=pl.BlockSpec((1, 1), lambda i: (0, 0)),
        out_shape=jax.ShapeDtypeStruct((1, 1), jnp.float32),
    )(x, y2)
    return out[0, 0]


# speedup vs baseline: 74.0803x; 1.3947x over previous
import jax
import jax.numpy as jnp
from jax import lax
from jax.experimental import pallas as pl
from jax.experimental.pallas import tpu as pltpu

_N_CLASSES = 100000
_K = 5
_BATCH = 128
_BR = 16


def _entr_loss_body(x_ref, y_ref, out_ref, losses_ref):
    rows = x_ref[...]
    yv = y_ref[...]
    col = lax.broadcasted_iota(jnp.int32, rows.shape, 1)

    fy = jnp.sum(jnp.where(col == yv, rows, 0.0), axis=1, keepdims=True)
    e_sum = jnp.sum(jnp.exp(rows), axis=1, keepdims=True)
    m1 = jnp.max(rows, axis=1, keepdims=True)

    neg_inf = jnp.float32(-jnp.inf)
    ms = [m1]
    for _ in range(_K - 1):
        ms.append(jnp.max(jnp.where(rows < ms[-1], rows, neg_inf),
                          axis=1, keepdims=True))
    m_last = ms[-1]

    cnt_ge = jnp.sum(jnp.where(rows >= m_last, 1.0, 0.0),
                     axis=1, keepdims=True)

    efy = jnp.exp(-fy)
    tops = jnp.zeros_like(fy)
    for m in ms:
        tops += jnp.exp(jnp.minimum(m - fy, 80.0))
    ind = jnp.where(fy < m_last, 1.0, 0.0)
    tail = efy * e_sum - tops - ind
    losses_ref[...] = jnp.log(1.0 + tail)

    @pl.when(jnp.any(cnt_ge != float(_K)))
    def _exact():
        rem = jnp.full_like(fy, float(_K))
        tops_x = jnp.zeros_like(fy)
        for m in ms:
            c = jnp.sum(jnp.where(rows == m, 1.0, 0.0), axis=1, keepdims=True)
            take = jnp.minimum(rem, c)
            tops_x += take * jnp.exp(jnp.minimum(m - fy, 80.0))
            rem -= take
        cnt_gt = jnp.sum(jnp.where(rows > fy, 1.0, 0.0), axis=1, keepdims=True)
        cnt_eqb = jnp.sum(jnp.where((rows == fy) & (col < yv), 1.0, 0.0),
                          axis=1, keepdims=True)
        ind_x = jnp.where(cnt_gt + cnt_eqb >= float(_K), 1.0, 0.0)
        tail_x = efy * e_sum - tops_x - ind_x
        losses_ref[...] = jnp.log(1.0 + tail_x)

    @pl.when(pl.program_id(0) == 0)
    def _init():
        out_ref[...] = jnp.zeros_like(out_ref)

    out_ref[...] += jnp.sum(losses_ref[...]).reshape(1, 1) * (1.0 / _BATCH)


@jax.jit
def kernel(x, y):
    grid = _BATCH // _BR
    y2 = y.reshape(_BATCH, 1)
    out = pl.pallas_call(
        _entr_loss_body,
        grid=(grid,),
        in_specs=[
            pl.BlockSpec((_BR, _N_CLASSES), lambda i: (i, 0)),
            pl.BlockSpec((_BR, 1), lambda i: (i, 0)),
        ],
        out_specs=pl.BlockSpec((1, 1), lambda i: (0, 0)),
        out_shape=jax.ShapeDtypeStruct((1, 1), jnp.float32),
        scratch_shapes=[pltpu.VMEM((_BR, 1), jnp.float32)],
    )(x, y2)
    return out[0, 0]


# speedup vs baseline: 119.1836x; 1.6088x over previous
import jax
import jax.numpy as jnp
from jax import lax
from jax.experimental import pallas as pl
from jax.experimental.pallas import tpu as pltpu

_N_CLASSES = 100000
_K = 5
_BATCH = 128
_BR = 8
_W = 1024
_NCH = _N_CLASSES // _W
_REM = _N_CLASSES - _NCH * _W


def _entr_loss_body(x_ref, y_ref, out_ref, losses_ref):
    yv = y_ref[...]
    neg_inf = jnp.float32(-jnp.inf)
    iota_w = lax.broadcasted_iota(jnp.int32, (_BR, _W), 1)

    a1 = jnp.full((_BR, _W), neg_inf)
    a2 = jnp.full((_BR, _W), neg_inf)
    a3 = jnp.full((_BR, _W), neg_inf)
    e_acc = jnp.zeros((_BR, _W), jnp.float32)
    fy_acc = jnp.zeros((_BR, _W), jnp.float32)
    for k in range(_NCH):
        t = x_ref[:, k * _W:(k + 1) * _W]
        e_acc += jnp.exp(t)
        fy_acc += jnp.where(iota_w == yv - (k * _W), t, 0.0)
        lo = jnp.minimum(a1, t)
        a1 = jnp.maximum(a1, t)
        lo2 = jnp.minimum(a2, lo)
        a2 = jnp.maximum(a2, lo)
        a3 = jnp.maximum(a3, lo2)

    rem = x_ref[:, _NCH * _W:]
    iota_r = lax.broadcasted_iota(jnp.int32, (_BR, _REM), 1)
    fy = (jnp.sum(fy_acc, axis=1, keepdims=True)
          + jnp.sum(jnp.where(iota_r == yv - (_NCH * _W), rem, 0.0),
                    axis=1, keepdims=True))
    e_sum = (jnp.sum(e_acc, axis=1, keepdims=True)
             + jnp.sum(jnp.exp(rem), axis=1, keepdims=True))

    cands = jnp.concatenate([a1, a2, a3, rem], axis=1)
    efy = jnp.exp(-fy)
    m = jnp.max(cands, axis=1, keepdims=True)
    v5 = m
    tops = jnp.zeros_like(fy)
    remaining = jnp.full_like(fy, float(_K))
    for t_i in range(_K):
        c = jnp.sum(jnp.where(cands == m, 1.0, 0.0), axis=1, keepdims=True)
        take = jnp.minimum(remaining, c)
        tops += take * jnp.exp(jnp.minimum(m - fy, 80.0))
        remaining -= take
        v5 = jnp.where(take > 0.0, m, v5)
        if t_i < _K - 1:
            m = jnp.max(jnp.where(cands < m, cands, neg_inf),
                        axis=1, keepdims=True)

    ind = jnp.where(fy < v5, 1.0, 0.0)
    tail = efy * e_sum - tops - ind
    losses_ref[...] = jnp.log(1.0 + tail)

    hid = jnp.any(a3 >= v5)
    tie = jnp.any(fy == v5)

    @pl.when(hid | tie)
    def _exact():
        rows = x_ref[...]
        col = lax.broadcasted_iota(jnp.int32, rows.shape, 1)
        rem5 = jnp.full_like(fy, float(_K))
        tops_x = jnp.zeros_like(fy)
        thr = jnp.full_like(fy, jnp.inf)
        for _ in range(_K):
            mx = jnp.max(jnp.where(rows < thr, rows, neg_inf),
                         axis=1, keepdims=True)
            cx = jnp.sum(jnp.where(rows == mx, 1.0, 0.0),
                         axis=1, keepdims=True)
            take_x = jnp.minimum(rem5, cx)
            tops_x += take_x * jnp.exp(jnp.minimum(mx - fy, 80.0))
            rem5 -= take_x
            thr = mx
        cnt_gt = jnp.sum(jnp.where(rows > fy, 1.0, 0.0),
                         axis=1, keepdims=True)
        cnt_eqb = jnp.sum(jnp.where((rows == fy) & (col < yv), 1.0, 0.0),
                          axis=1, keepdims=True)
        ind_x = jnp.where(cnt_gt + cnt_eqb >= float(_K), 1.0, 0.0)
        tail_x = efy * e_sum - tops_x - ind_x
        losses_ref[...] = jnp.log(1.0 + tail_x)

    @pl.when(pl.program_id(0) == 0)
    def _init():
        out_ref[...] = jnp.zeros_like(out_ref)

    out_ref[...] += jnp.sum(losses_ref[...]).reshape(1, 1) * (1.0 / _BATCH)


@jax.jit
def kernel(x, y):
    grid = _BATCH // _BR
    y2 = y.reshape(_BATCH, 1)
    out = pl.pallas_call(
        _entr_loss_body,
        grid=(grid,),
        in_specs=[
            pl.BlockSpec((_BR, _N_CLASSES), lambda i: (i, 0)),
            pl.BlockSpec((_BR, 1), lambda i: (i, 0)),
        ],
        out_specs=pl.BlockSpec((1, 1), lambda i: (0, 0)),
        out_shape=jax.ShapeDtypeStruct((1, 1), jnp.float32),
        scratch_shapes=[pltpu.VMEM((_BR, 1), jnp.float32)],
    )(x, y2)
    return out[0, 0]


# speedup vs baseline: 119.4446x; 1.0022x over previous
import jax
import jax.numpy as jnp
from jax import lax
from jax.experimental import pallas as pl
from jax.experimental.pallas import tpu as pltpu

_N_CLASSES = 100000
_K = 5
_BATCH = 128
_BR = 8
_W = 1024
_NCH = _N_CLASSES // _W
_REM = _N_CLASSES - _NCH * _W


def _entr_loss_body(x_ref, y_ref, out_ref, losses_ref):
    yv = y_ref[pl.ds(pl.program_id(0) * _BR, _BR), :]
    neg_inf = jnp.float32(-jnp.inf)
    iota_w = lax.broadcasted_iota(jnp.int32, (_BR, _W), 1)

    a1 = jnp.full((_BR, _W), neg_inf)
    a2 = jnp.full((_BR, _W), neg_inf)
    a3 = jnp.full((_BR, _W), neg_inf)
    e_acc = jnp.zeros((_BR, _W), jnp.float32)
    fy_acc = jnp.zeros((_BR, _W), jnp.float32)
    for k in range(_NCH):
        t = x_ref[:, k * _W:(k + 1) * _W]
        e_acc += jnp.exp(t)
        fy_acc += jnp.where(iota_w == yv - (k * _W), t, 0.0)
        lo = jnp.minimum(a1, t)
        a1 = jnp.maximum(a1, t)
        lo2 = jnp.minimum(a2, lo)
        a2 = jnp.maximum(a2, lo)
        a3 = jnp.maximum(a3, lo2)

    rem = x_ref[:, _NCH * _W:]
    iota_r = lax.broadcasted_iota(jnp.int32, (_BR, _REM), 1)
    fy = (jnp.sum(fy_acc, axis=1, keepdims=True)
          + jnp.sum(jnp.where(iota_r == yv - (_NCH * _W), rem, 0.0),
                    axis=1, keepdims=True))
    e_sum = (jnp.sum(e_acc, axis=1, keepdims=True)
             + jnp.sum(jnp.exp(rem), axis=1, keepdims=True))

    cands = jnp.concatenate([a1, a2, a3, rem], axis=1)
    efy = jnp.exp(-fy)
    m = jnp.max(cands, axis=1, keepdims=True)
    v5 = m
    tops = jnp.zeros_like(fy)
    remaining = jnp.full_like(fy, float(_K))
    for t_i in range(_K):
        c = jnp.sum(jnp.where(cands == m, 1.0, 0.0), axis=1, keepdims=True)
        take = jnp.minimum(remaining, c)
        tops += take * jnp.exp(jnp.minimum(m - fy, 80.0))
        remaining -= take
        v5 = jnp.where(take > 0.0, m, v5)
        if t_i < _K - 1:
            m = jnp.max(jnp.where(cands < m, cands, neg_inf),
                        axis=1, keepdims=True)

    ind = jnp.where(fy < v5, 1.0, 0.0)
    tail = efy * e_sum - tops - ind
    losses_ref[...] = jnp.log(1.0 + tail)

    hid = jnp.any(a3 >= v5)
    tie = jnp.any(fy == v5)

    @pl.when(hid | tie)
    def _exact():
        rows = x_ref[...]
        col = lax.broadcasted_iota(jnp.int32, rows.shape, 1)
        rem5 = jnp.full_like(fy, float(_K))
        tops_x = jnp.zeros_like(fy)
        thr = jnp.full_like(fy, jnp.inf)
        for _ in range(_K):
            mx = jnp.max(jnp.where(rows < thr, rows, neg_inf),
                         axis=1, keepdims=True)
            cx = jnp.sum(jnp.where(rows == mx, 1.0, 0.0),
                         axis=1, keepdims=True)
            take_x = jnp.minimum(rem5, cx)
            tops_x += take_x * jnp.exp(jnp.minimum(mx - fy, 80.0))
            rem5 -= take_x
            thr = mx
        cnt_gt = jnp.sum(jnp.where(rows > fy, 1.0, 0.0),
                         axis=1, keepdims=True)
        cnt_eqb = jnp.sum(jnp.where((rows == fy) & (col < yv), 1.0, 0.0),
                          axis=1, keepdims=True)
        ind_x = jnp.where(cnt_gt + cnt_eqb >= float(_K), 1.0, 0.0)
        tail_x = efy * e_sum - tops_x - ind_x
        losses_ref[...] = jnp.log(1.0 + tail_x)

    @pl.when(pl.program_id(0) == 0)
    def _init():
        out_ref[...] = jnp.zeros_like(out_ref)

    out_ref[...] += jnp.sum(losses_ref[...]).reshape(1, 1) * (1.0 / _BATCH)


@jax.jit
def kernel(x, y):
    grid = _BATCH // _BR
    y2 = y.reshape(_BATCH, 1)
    out = pl.pallas_call(
        _entr_loss_body,
        grid=(grid,),
        in_specs=[
            pl.BlockSpec((_BR, _N_CLASSES), lambda i: (i, 0)),
            pl.BlockSpec((_BATCH, 1), lambda i: (0, 0)),
        ],
        out_specs=pl.BlockSpec((1, 1), lambda i: (0, 0)),
        out_shape=jax.ShapeDtypeStruct((1, 1), jnp.float32),
        scratch_shapes=[pltpu.VMEM((_BR, 1), jnp.float32)],
    )(x, y2)
    return out[0, 0]


# speedup vs baseline: 159.5591x; 1.3358x over previous
import jax
import jax.numpy as jnp
from jax import lax
from jax.experimental import pallas as pl
from jax.experimental.pallas import tpu as pltpu

_N_CLASSES = 100000
_K = 5
_BATCH = 128
_BC = 10000
_S = 40
_NSL = _BC // _S
_GRID = _N_CLASSES // _BC
_NEG = float(-jnp.inf)


def _stream_body(xt_ref, y_ref, loss_ref, tie_ref, a_ref, es_ref, fy_ref):
    k = pl.program_id(0)

    @pl.when(k == 0)
    def _init():
        a_ref[...] = jnp.full_like(a_ref, _NEG)
        es_ref[...] = jnp.zeros_like(es_ref)
        fy_ref[...] = jnp.zeros_like(fy_ref)

    yv = y_ref[...]
    iota_s = lax.broadcasted_iota(jnp.int32, (_S, _BATCH), 0)
    base0 = k * _BC

    def slab(j, c):
        a1, a2, a3, a4, a5, es, fy = c
        t = xt_ref[pl.ds(j * _S, _S), :]
        es = es + jnp.exp(t)
        fy = fy + jnp.where(iota_s == yv - (base0 + j * _S), t, 0.0)
        lo = jnp.minimum(a1, t)
        a1 = jnp.maximum(a1, t)
        lo2 = jnp.minimum(a2, lo)
        a2 = jnp.maximum(a2, lo)
        lo3 = jnp.minimum(a3, lo2)
        a3 = jnp.maximum(a3, lo2)
        lo4 = jnp.minimum(a4, lo3)
        a4 = jnp.maximum(a4, lo3)
        a5 = jnp.maximum(a5, lo4)
        return a1, a2, a3, a4, a5, es, fy

    carry = (a_ref[0], a_ref[1], a_ref[2], a_ref[3], a_ref[4],
             es_ref[...], fy_ref[...])
    carry = lax.fori_loop(0, _NSL, slab, carry)
    for i in range(5):
        a_ref[i] = carry[i]
    es_ref[...] = carry[5]
    fy_ref[...] = carry[6]

    @pl.when(k == _GRID - 1)
    def _finish():
        accs = [a_ref[i] for i in range(5)]
        es_l = jnp.sum(es_ref[...], axis=0, keepdims=True)
        fy_l = jnp.sum(fy_ref[...], axis=0, keepdims=True)

        neg_inf = jnp.float32(_NEG)
        m = accs[0][0:1, :]
        for a in accs:
            m = jnp.maximum(m, jnp.max(a, axis=0, keepdims=True))
        tops = jnp.zeros_like(fy_l)
        remaining = jnp.full_like(fy_l, float(_K))
        v5 = m
        for t_i in range(_K):
            c = jnp.zeros_like(fy_l)
            for a in accs:
                c += jnp.sum(jnp.where(a == m, 1.0, 0.0), axis=0,
                             keepdims=True)
            take = jnp.minimum(remaining, c)
            tops += take * jnp.exp(jnp.minimum(m - fy_l, 80.0))
            remaining -= take
            v5 = jnp.where(take > 0.0, m, v5)
            if t_i < _K - 1:
                nm = jnp.full_like(fy_l, _NEG)
                for a in accs:
                    nm = jnp.maximum(
                        nm, jnp.max(jnp.where(a < m, a, neg_inf),
                                    axis=0, keepdims=True))
                m = nm

        ind = jnp.where(fy_l < v5, 1.0, 0.0)
        tail = jnp.exp(-fy_l) * es_l - tops - ind
        losses = jnp.log(1.0 + tail)
        loss_ref[...] = (jnp.sum(losses) * (1.0 / _BATCH)).reshape(1, 1)
        tie = jnp.any(fy_l == v5)
        tie_ref[...] = jnp.where(tie, 1.0, 0.0).reshape(1, 1)


def _exact_body(x_ref, y_ref, out_ref, losses_ref):
    _BR = 8
    yv = y_ref[pl.ds(pl.program_id(0) * _BR, _BR), :]
    rows = x_ref[...]
    col = lax.broadcasted_iota(jnp.int32, rows.shape, 1)
    neg_inf = jnp.float32(_NEG)

    fy = jnp.sum(jnp.where(col == yv, rows, 0.0), axis=1, keepdims=True)
    e_sum = jnp.sum(jnp.exp(rows), axis=1, keepdims=True)
    rem5 = jnp.full_like(fy, float(_K))
    tops = jnp.zeros_like(fy)
    thr = jnp.full_like(fy, jnp.inf)
    for _ in range(_K):
        mx = jnp.max(jnp.where(rows < thr, rows, neg_inf),
                     axis=1, keepdims=True)
        cx = jnp.sum(jnp.where(rows == mx, 1.0, 0.0), axis=1, keepdims=True)
        take = jnp.minimum(rem5, cx)
        tops += take * jnp.exp(jnp.minimum(mx - fy, 80.0))
        rem5 -= take
        thr = mx
    cnt_gt = jnp.sum(jnp.where(rows > fy, 1.0, 0.0), axis=1, keepdims=True)
    cnt_eqb = jnp.sum(jnp.where((rows == fy) & (col < yv), 1.0, 0.0),
                      axis=1, keepdims=True)
    ind = jnp.where(cnt_gt + cnt_eqb >= float(_K), 1.0, 0.0)
    tail = jnp.exp(-fy) * e_sum - tops - ind
    losses_ref[...] = jnp.log(1.0 + tail)

    @pl.when(pl.program_id(0) == 0)
    def _init():
        out_ref[...] = jnp.zeros_like(out_ref)

    out_ref[...] += jnp.sum(losses_ref[...]).reshape(1, 1) * (1.0 / _BATCH)


def _exact_loss(x, y2):
    return pl.pallas_call(
        _exact_body,
        grid=(16,),
        in_specs=[
            pl.BlockSpec((8, _N_CLASSES), lambda i: (i, 0)),
            pl.BlockSpec((_BATCH, 1), lambda i: (0, 0)),
        ],
        out_specs=pl.BlockSpec((1, 1), lambda i: (0, 0)),
        out_shape=jax.ShapeDtypeStruct((1, 1), jnp.float32),
        scratch_shapes=[pltpu.VMEM((8, 1), jnp.float32)],
    )(x, y2)[0, 0]


@jax.jit
def kernel(x, y):
    xt = x.T
    yr = y.reshape(1, _BATCH)
    loss, tie = pl.pallas_call(
        _stream_body,
        grid=(_GRID,),
        in_specs=[
            pl.BlockSpec((_BC, _BATCH), lambda i: (i, 0)),
            pl.BlockSpec((1, _BATCH), lambda i: (0, 0)),
        ],
        out_specs=[
            pl.BlockSpec((1, 1), lambda i: (0, 0)),
            pl.BlockSpec((1, 1), lambda i: (0, 0)),
        ],
        out_shape=[
            jax.ShapeDtypeStruct((1, 1), jnp.float32),
            jax.ShapeDtypeStruct((1, 1), jnp.float32),
        ],
        scratch_shapes=[
            pltpu.VMEM((5, _S, _BATCH), jnp.float32),
            pltpu.VMEM((_S, _BATCH), jnp.float32),
            pltpu.VMEM((_S, _BATCH), jnp.float32),
        ],
    )(xt, yr)
    y2 = y.reshape(_BATCH, 1)
    return lax.cond(tie[0, 0] > 0.0,
                    lambda ops: _exact_loss(*ops),
                    lambda ops: loss[0, 0],
                    (x, y2))


# speedup vs baseline: 262.5374x; 1.6454x over previous
import jax
import jax.numpy as jnp
from jax import lax
from jax.experimental import pallas as pl
from jax.experimental.pallas import tpu as pltpu

_N_CLASSES = 100000
_K = 5
_BATCH = 128
_BC = 10000
_S = 40
_NSL = _BC // _S
_GRID = _N_CLASSES // _BC
_NEG = float(-jnp.inf)


def _stream_body(xt_ref, y_ref, loss_ref, tie_ref, a_ref, es_ref, fy_ref):
    k = pl.program_id(0)

    @pl.when(k == 0)
    def _init():
        a_ref[...] = jnp.full_like(a_ref, _NEG)
        es_ref[...] = jnp.zeros_like(es_ref)
        fy_ref[...] = jnp.zeros_like(fy_ref)

    yv = y_ref[...]
    iota_s = lax.broadcasted_iota(jnp.int32, (_S, _BATCH), 0)
    yvb = yv + jnp.zeros((_S, _BATCH), jnp.int32)
    base0 = k * _BC

    def slab(j, c):
        a1, a2, a3, a4, a5, es, fy = c
        t = xt_ref[pl.ds(j * _S, _S), :]
        es = es + jnp.exp(t)
        fy = fy + jnp.where(iota_s + (base0 + j * _S) == yvb, t, 0.0)
        lo = jnp.minimum(a1, t)
        a1 = jnp.maximum(a1, t)
        lo2 = jnp.minimum(a2, lo)
        a2 = jnp.maximum(a2, lo)
        lo3 = jnp.minimum(a3, lo2)
        a3 = jnp.maximum(a3, lo2)
        lo4 = jnp.minimum(a4, lo3)
        a4 = jnp.maximum(a4, lo3)
        a5 = jnp.maximum(a5, lo4)
        return a1, a2, a3, a4, a5, es, fy

    carry = (a_ref[0], a_ref[1], a_ref[2], a_ref[3], a_ref[4],
             es_ref[...], fy_ref[...])
    carry = lax.fori_loop(0, _NSL, slab, carry, unroll=5)
    for i in range(5):
        a_ref[i] = carry[i]
    es_ref[...] = carry[5]
    fy_ref[...] = carry[6]

    @pl.when(k == _GRID - 1)
    def _finish():
        accs = [a_ref[i] for i in range(5)]
        es_l = jnp.sum(es_ref[...], axis=0, keepdims=True)
        fy_l = jnp.sum(fy_ref[...], axis=0, keepdims=True)

        neg_inf = jnp.float32(_NEG)
        m = accs[0][0:1, :]
        for a in accs:
            m = jnp.maximum(m, jnp.max(a, axis=0, keepdims=True))
        tops = jnp.zeros_like(fy_l)
        remaining = jnp.full_like(fy_l, float(_K))
        v5 = m
        for t_i in range(_K):
            c = jnp.zeros_like(fy_l)
            for a in accs:
                c += jnp.sum(jnp.where(a == m, 1.0, 0.0), axis=0,
                             keepdims=True)
            take = jnp.minimum(remaining, c)
            tops += take * jnp.exp(jnp.minimum(m - fy_l, 80.0))
            remaining -= take
            v5 = jnp.where(take > 0.0, m, v5)
            if t_i < _K - 1:
                nm = jnp.full_like(fy_l, _NEG)
                for a in accs:
                    nm = jnp.maximum(
                        nm, jnp.max(jnp.where(a < m, a, neg_inf),
                                    axis=0, keepdims=True))
                m = nm

        ind = jnp.where(fy_l < v5, 1.0, 0.0)
        tail = jnp.exp(-fy_l) * es_l - tops - ind
        losses = jnp.log(1.0 + tail)
        loss_ref[...] = (jnp.sum(losses) * (1.0 / _BATCH)).reshape(1, 1)
        tie = jnp.any(fy_l == v5)
        tie_ref[...] = jnp.where(tie, 1.0, 0.0).reshape(1, 1)


def _exact_body(x_ref, y_ref, out_ref, losses_ref):
    _BR = 8
    yv = y_ref[pl.ds(pl.program_id(0) * _BR, _BR), :]
    rows = x_ref[...]
    col = lax.broadcasted_iota(jnp.int32, rows.shape, 1)
    neg_inf = jnp.float32(_NEG)

    fy = jnp.sum(jnp.where(col == yv, rows, 0.0), axis=1, keepdims=True)
    e_sum = jnp.sum(jnp.exp(rows), axis=1, keepdims=True)
    rem5 = jnp.full_like(fy, float(_K))
    tops = jnp.zeros_like(fy)
    thr = jnp.full_like(fy, jnp.inf)
    for _ in range(_K):
        mx = jnp.max(jnp.where(rows < thr, rows, neg_inf),
                     axis=1, keepdims=True)
        cx = jnp.sum(jnp.where(rows == mx, 1.0, 0.0), axis=1, keepdims=True)
        take = jnp.minimum(rem5, cx)
        tops += take * jnp.exp(jnp.minimum(mx - fy, 80.0))
        rem5 -= take
        thr = mx
    cnt_gt = jnp.sum(jnp.where(rows > fy, 1.0, 0.0), axis=1, keepdims=True)
    cnt_eqb = jnp.sum(jnp.where((rows == fy) & (col < yv), 1.0, 0.0),
                      axis=1, keepdims=True)
    ind = jnp.where(cnt_gt + cnt_eqb >= float(_K), 1.0, 0.0)
    tail = jnp.exp(-fy) * e_sum - tops - ind
    losses_ref[...] = jnp.log(1.0 + tail)

    @pl.when(pl.program_id(0) == 0)
    def _init():
        out_ref[...] = jnp.zeros_like(out_ref)

    out_ref[...] += jnp.sum(losses_ref[...]).reshape(1, 1) * (1.0 / _BATCH)


def _exact_loss(x, y2):
    return pl.pallas_call(
        _exact_body,
        grid=(16,),
        in_specs=[
            pl.BlockSpec((8, _N_CLASSES), lambda i: (i, 0)),
            pl.BlockSpec((_BATCH, 1), lambda i: (0, 0)),
        ],
        out_specs=pl.BlockSpec((1, 1), lambda i: (0, 0)),
        out_shape=jax.ShapeDtypeStruct((1, 1), jnp.float32),
        scratch_shapes=[pltpu.VMEM((8, 1), jnp.float32)],
    )(x, y2)[0, 0]


@jax.jit
def kernel(x, y):
    xt = x.T
    yr = y.reshape(1, _BATCH)
    loss, tie = pl.pallas_call(
        _stream_body,
        grid=(_GRID,),
        in_specs=[
            pl.BlockSpec((_BC, _BATCH), lambda i: (i, 0)),
            pl.BlockSpec((1, _BATCH), lambda i: (0, 0)),
        ],
        out_specs=[
            pl.BlockSpec((1, 1), lambda i: (0, 0)),
            pl.BlockSpec((1, 1), lambda i: (0, 0)),
        ],
        out_shape=[
            jax.ShapeDtypeStruct((1, 1), jnp.float32),
            jax.ShapeDtypeStruct((1, 1), jnp.float32),
        ],
        scratch_shapes=[
            pltpu.VMEM((5, _S, _BATCH), jnp.float32),
            pltpu.VMEM((_S, _BATCH), jnp.float32),
            pltpu.VMEM((_S, _BATCH), jnp.float32),
        ],
    )(xt, yr)
    y2 = y.reshape(_BATCH, 1)
    return lax.cond(tie[0, 0] > 0.0,
                    lambda ops: _exact_loss(*ops),
                    lambda ops: loss[0, 0],
                    (x, y2))
